# Initial kernel scaffold; baseline (speedup 1.0000x reference)
import jax
import jax.numpy as jnp
from jax.experimental import pallas as pl


def kernel(x, edge_index, W_enc, b_enc, W_gc, b_gc, W_dec, b_dec):
    raise NotImplementedError("write your pallas kernel here")


# speedup vs baseline: 12.1820x; 12.1820x over previous
import functools

import jax
import jax.numpy as jnp
from jax import lax
from jax.experimental import pallas as pl
from jax.experimental.pallas import tpu as pltpu
from jax.experimental.pallas import tpu_sc as plsc

N = 10000
NP = 10240
E = 320000
D_IN = 128
D_H = 256
D_OUT = 128
NSUB = 16
CHUNK = 128
K = 157
EP = NSUB * K * CHUNK
ROWS_PER_TILE = NP // NSUB
R = 512
GRID = NP // R
SCHEDULE = (0.5, 0.5, 0.5, 0.5)
DEG_W = 16

_mesh = plsc.VectorSubcoreMesh(core_axis_name="c", subcore_axis_name="s")


@functools.partial(
    pl.kernel,
    mesh=_mesh,
    out_type=jax.ShapeDtypeStruct((NP, DEG_W), jnp.float32),
    scratch_types=[
        pltpu.VMEM((K, CHUNK), jnp.int32),
        pltpu.VMEM((CHUNK, DEG_W), jnp.float32),
        pltpu.VMEM((ROWS_PER_TILE, DEG_W), jnp.float32),
        pltpu.VMEM_SHARED((NP, DEG_W), jnp.float32),
    ],
    compiler_params=pltpu.CompilerParams(use_tc_tiling_on_sc=False),
)
def _deg_kernel(dst_hbm, out_hbm, dst_v, ones_v, zero_v, hist):
    c = lax.axis_index("c")
    s = lax.axis_index("s")
    pltpu.sync_copy(dst_hbm.at[s, :, :], dst_v)

    def fill_ones(i, carry):
        ones_v[i] = jnp.ones((DEG_W,), jnp.float32)
        return carry

    lax.fori_loop(0, CHUNK, fill_ones, 0)

    def fill_zero(i, carry):
        zero_v[i] = jnp.zeros((DEG_W,), jnp.float32)
        return carry

    lax.fori_loop(0, ROWS_PER_TILE, fill_zero, 0)
    pltpu.sync_copy(zero_v, hist.at[pl.ds(s * ROWS_PER_TILE, ROWS_PER_TILE)])
    plsc.subcore_barrier()

    def step(j, carry):
        pltpu.sync_copy(ones_v, hist.at[dst_v.at[j]], add=True)
        return carry

    lax.fori_loop(0, K, step, 0)
    plsc.subcore_barrier()
    half = NP // 2
    per = half // NSUB
    wb = c * half + s * per
    pltpu.sync_copy(hist.at[pl.ds(wb, per)], out_hbm.at[pl.ds(wb, per)])


QW = 64


@functools.partial(
    pl.kernel,
    mesh=_mesh,
    out_type=jax.ShapeDtypeStruct((4 * NP, QW), jnp.float32),
    scratch_types=[
        pltpu.VMEM((K, CHUNK), jnp.int32),
        pltpu.VMEM((K, CHUNK), jnp.int32),
        pltpu.VMEM((K, CHUNK), jnp.int32),
        pltpu.VMEM((CHUNK, QW), jnp.float32),
        pltpu.VMEM((CHUNK, QW), jnp.float32),
        pltpu.VMEM_SHARED((NP, QW), jnp.float32),
        pltpu.SemaphoreType.DMA,
        pltpu.SemaphoreType.DMA,
    ],
    compiler_params=pltpu.CompilerParams(use_tc_tiling_on_sc=False),
)
def _agg_kernel(y_hbm, src_hbm, dst_hbm, out_hbm, src_v0, src_v1, dst_v,
                rows_a, rows_b, acc, sem_a, sem_b):
    c = lax.axis_index("c")
    s = lax.axis_index("s")
    pltpu.sync_copy(src_hbm.at[c, 0, s, :, :], src_v0)
    pltpu.sync_copy(src_hbm.at[c, 1, s, :, :], src_v1)
    pltpu.sync_copy(dst_hbm.at[s, :, :], dst_v)
    base = s * ROWS_PER_TILE

    for p, src_v in ((0, src_v0), (1, src_v1)):
        off = (2 * c + p) * NP + base
        pltpu.sync_copy(y_hbm.at[pl.ds(off, ROWS_PER_TILE)],
                        acc.at[pl.ds(base, ROWS_PER_TILE)])
        plsc.subcore_barrier()

        def gather_start(j, buf, sem):
            pltpu.async_copy(y_hbm.at[src_v.at[j]], buf, sem)

        def gather_wait(j, buf, sem):
            pltpu.make_async_copy(y_hbm.at[src_v.at[j]], buf, sem).wait()

        gather_start(0, rows_a, sem_a)

        def pair(q, carry):
            j = 2 * q
            gather_start(j + 1, rows_b, sem_b)
            gather_wait(j, rows_a, sem_a)
            pltpu.sync_copy(rows_a, acc.at[dst_v.at[j]], add=True)
            gather_start(j + 2, rows_a, sem_a)
            gather_wait(j + 1, rows_b, sem_b)
            pltpu.sync_copy(rows_b, acc.at[dst_v.at[j + 1]], add=True)
            return carry

        lax.fori_loop(0, K // 2, pair, 0)
        gather_wait(K - 1, rows_a, sem_a)
        pltpu.sync_copy(rows_a, acc.at[dst_v.at[K - 1]], add=True)
        plsc.subcore_barrier()
        pltpu.sync_copy(acc.at[pl.ds(base, ROWS_PER_TILE)],
                        out_hbm.at[pl.ds(off, ROWS_PER_TILE)])


def _dis_block(deg_blk, block_id):
    cnt = jnp.sum(deg_blk, axis=1, keepdims=True) * (1.0 / DEG_W) + 1.0
    rows = lax.broadcasted_iota(jnp.int32, (R, 1), 0) + block_id * R
    return jnp.where(rows < N, lax.rsqrt(cnt), 0.0)


def _tc_enc_body(t_first, x_ref, we_ref, be_ref, wm_ref, wt_ref, deg_ref,
                 h_ref, y_ref):
    b = pl.program_id(0)
    h = jnp.maximum(
        jnp.dot(x_ref[...], we_ref[...], preferred_element_type=jnp.float32)
        + be_ref[...], 0.0)
    h_ref[...] = h
    dis = _dis_block(deg_ref[...], b)
    y = (jnp.dot(h, wm_ref[...], preferred_element_type=jnp.float32)
         + t_first * wt_ref[...]) * dis
    for q in range(4):
        y_ref[q] = y[:, q * QW:(q + 1) * QW]


def _tc_mid_body(t_next, h_ref, agg_ref, deg_ref, bg_ref, wm_ref, wt_ref,
                 ho_ref, yo_ref):
    b = pl.program_id(0)
    dis = _dis_block(deg_ref[...], b)
    aggcat = jnp.concatenate([agg_ref[q] for q in range(4)], axis=1)
    conv = dis * aggcat + bg_ref[...]
    hn = 0.5 * h_ref[...] + 0.5 * jnp.maximum(conv, 0.0)
    ho_ref[...] = hn
    y = (jnp.dot(hn, wm_ref[...], preferred_element_type=jnp.float32)
         + t_next * wt_ref[...]) * dis
    for q in range(4):
        yo_ref[q] = y[:, q * QW:(q + 1) * QW]


def _tc_last_body(h_ref, agg_ref, deg_ref, bg_ref, wd_ref, bd_ref, o_ref):
    b = pl.program_id(0)
    dis = _dis_block(deg_ref[...], b)
    aggcat = jnp.concatenate([agg_ref[q] for q in range(4)], axis=1)
    conv = dis * aggcat + bg_ref[...]
    hn = 0.5 * h_ref[...] + 0.5 * jnp.maximum(conv, 0.0)
    o_ref[...] = (jnp.dot(hn, wd_ref[...], preferred_element_type=jnp.float32)
                  + bd_ref[...])


_ROWB = pl.BlockSpec((R, D_H), lambda b: (b, 0))
_ROWB128 = pl.BlockSpec((R, 128), lambda b: (b, 0))
_Y_B = pl.BlockSpec((4, R, QW), lambda b: (0, b, 0))
_DEG_B = pl.BlockSpec((R, DEG_W), lambda b: (b, 0))
_FULL = lambda shape: pl.BlockSpec(shape, lambda b: tuple(0 for _ in shape))


def _tc_enc(t_first):
    return pl.pallas_call(
        functools.partial(_tc_enc_body, t_first),
        grid=(GRID,),
        in_specs=[_ROWB128, _FULL((D_IN, D_H)), _FULL((1, D_H)),
                  _FULL((D_H, D_H)), _FULL((1, D_H)), _DEG_B],
        out_specs=[_ROWB, _Y_B],
        out_shape=[jax.ShapeDtypeStruct((NP, D_H), jnp.float32),
                   jax.ShapeDtypeStruct((4, NP, QW), jnp.float32)],
    )


def _tc_mid(t_next):
    return pl.pallas_call(
        functools.partial(_tc_mid_body, t_next),
        grid=(GRID,),
        in_specs=[_ROWB, _Y_B, _DEG_B, _FULL((1, D_H)),
                  _FULL((D_H, D_H)), _FULL((1, D_H))],
        out_specs=[_ROWB, _Y_B],
        out_shape=[jax.ShapeDtypeStruct((NP, D_H), jnp.float32),
                   jax.ShapeDtypeStruct((4, NP, QW), jnp.float32)],
    )


_tc_last = pl.pallas_call(
    _tc_last_body,
    grid=(GRID,),
    in_specs=[_ROWB, _Y_B, _DEG_B, _FULL((1, D_H)),
              _FULL((D_H, D_OUT)), _FULL((1, D_OUT))],
    out_specs=_ROWB128,
    out_shape=jax.ShapeDtypeStruct((NP, D_OUT), jnp.float32),
)


def kernel(x, edge_index, W_enc, b_enc, W_gc, b_gc, W_dec, b_dec):
    src = edge_index[0].astype(jnp.int32)
    dst = edge_index[1].astype(jnp.int32)
    pad = jnp.full((EP - E,), N, jnp.int32)
    src_p = jnp.concatenate([src, pad])
    dst_p = jnp.concatenate([dst, pad]).reshape(NSUB, K, CHUNK)
    src_4 = jnp.stack([src_p, src_p + NP, src_p + 2 * NP, src_p + 3 * NP])
    src_4 = src_4.reshape(2, 2, NSUB, K, CHUNK)

    deg16 = _deg_kernel(dst_p)

    x_p = jnp.pad(x, ((0, NP - N), (0, 0)))
    Wm = W_gc[:D_H]
    wt = W_gc[D_H:D_H + 1]
    be = b_enc.reshape(1, D_H)
    bg = b_gc.reshape(1, D_H)
    bd = b_dec.reshape(1, D_OUT)

    num_iter = len(SCHEDULE)
    h, y = _tc_enc(1.0 / num_iter)(x_p, W_enc, be, Wm, wt, deg16)
    for it in range(1, num_iter):
        agg = _agg_kernel(y.reshape(4 * NP, QW), src_4, dst_p)
        h, y = _tc_mid((it + 1.0) / num_iter)(
            h, agg.reshape(4, NP, QW), deg16, bg, Wm, wt)
    agg = _agg_kernel(y.reshape(4 * NP, QW), src_4, dst_p)
    out = _tc_last(h, agg.reshape(4, NP, QW), deg16, bg, W_dec, bd)
    return out[:N]


# speedup vs baseline: 16.5723x; 1.3604x over previous
import functools

import jax
import jax.numpy as jnp
from jax import lax
from jax.experimental import pallas as pl
from jax.experimental.pallas import tpu as pltpu
from jax.experimental.pallas import tpu_sc as plsc

N = 10000
NP = 10240
E = 320000
D_IN = 128
D_H = 256
D_OUT = 128
NSUB = 16
CHUNK = 128
K = 162
EP = NSUB * K * CHUNK
ROWS_PER_TILE = NP // NSUB
R = 512
GRID = NP // R
SCHEDULE = (0.5, 0.5, 0.5, 0.5)
DEG_W = 16

_mesh = plsc.VectorSubcoreMesh(core_axis_name="c", subcore_axis_name="s")


@functools.partial(
    pl.kernel,
    mesh=_mesh,
    out_type=jax.ShapeDtypeStruct((NP, DEG_W), jnp.float32),
    scratch_types=[
        pltpu.VMEM((K, CHUNK), jnp.int32),
        pltpu.VMEM((CHUNK, DEG_W), jnp.float32),
        pltpu.VMEM((ROWS_PER_TILE, DEG_W), jnp.float32),
        pltpu.VMEM_SHARED((NP, DEG_W), jnp.float32),
    ],
    compiler_params=pltpu.CompilerParams(use_tc_tiling_on_sc=False),
)
def _deg_kernel(dst_hbm, out_hbm, dst_v, ones_v, zero_v, hist):
    c = lax.axis_index("c")
    s = lax.axis_index("s")
    pltpu.sync_copy(dst_hbm.at[s, :, :], dst_v)

    def fill_ones(i, carry):
        ones_v[i] = jnp.ones((DEG_W,), jnp.float32)
        return carry

    lax.fori_loop(0, CHUNK, fill_ones, 0)

    def fill_zero(i, carry):
        zero_v[i] = jnp.zeros((DEG_W,), jnp.float32)
        return carry

    lax.fori_loop(0, ROWS_PER_TILE, fill_zero, 0)
    pltpu.sync_copy(zero_v, hist.at[pl.ds(s * ROWS_PER_TILE, ROWS_PER_TILE)])
    plsc.subcore_barrier()

    def step(j, carry):
        pltpu.sync_copy(ones_v, hist.at[dst_v.at[j]], add=True)
        return carry

    lax.fori_loop(0, K, step, 0)
    plsc.subcore_barrier()
    half = NP // 2
    per = half // NSUB
    wb = c * half + s * per
    pltpu.sync_copy(hist.at[pl.ds(wb, per)], out_hbm.at[pl.ds(wb, per)])


QW = 64


NBUF = 3


@functools.partial(
    pl.kernel,
    mesh=_mesh,
    out_type=jax.ShapeDtypeStruct((4 * NP, QW), jnp.float32),
    scratch_types=[
        pltpu.VMEM((K, CHUNK), jnp.int32),
        pltpu.VMEM((K, CHUNK), jnp.int32),
        pltpu.VMEM((K, CHUNK), jnp.int32),
        [pltpu.VMEM((CHUNK, QW), jnp.float32) for _ in range(NBUF)],
        pltpu.VMEM_SHARED((NP, QW), jnp.float32),
        [pltpu.SemaphoreType.DMA for _ in range(NBUF)],
        [pltpu.SemaphoreType.DMA for _ in range(NBUF)],
    ],
    compiler_params=pltpu.CompilerParams(use_tc_tiling_on_sc=False),
)
def _agg_kernel(y_hbm, src_hbm, dst_hbm, out_hbm, src_v0, src_v1, dst_v,
                rows, acc, gsem, ssem):
    c = lax.axis_index("c")
    s = lax.axis_index("s")
    pltpu.sync_copy(src_hbm.at[c, 0, s, :, :], src_v0)
    pltpu.sync_copy(src_hbm.at[c, 1, s, :, :], src_v1)
    pltpu.sync_copy(dst_hbm.at[s, :, :], dst_v)
    base = s * ROWS_PER_TILE

    for p, src_v in ((0, src_v0), (1, src_v1)):
        off = (2 * c + p) * NP + base
        pltpu.sync_copy(y_hbm.at[pl.ds(off, ROWS_PER_TILE)],
                        acc.at[pl.ds(base, ROWS_PER_TILE)])
        plsc.subcore_barrier()

        def gather_start(j, l):
            pltpu.async_copy(y_hbm.at[src_v.at[j]], rows[l], gsem[l])

        def gather_wait(j, l):
            pltpu.make_async_copy(y_hbm.at[src_v.at[j]], rows[l], gsem[l]).wait()

        def scatter_start(j, l):
            pltpu.async_copy(rows[l], acc.at[dst_v.at[j]], ssem[l], add=True)

        def scatter_wait(j, l):
            pltpu.make_async_copy(rows[l], acc.at[dst_v.at[j]], ssem[l]).wait()

        for l in range(NBUF - 1):
            gather_start(l, l)

        def group(g, carry):
            for l in range(NBUF):
                j = NBUF * g + l
                gather_wait(j, l)
                scatter_start(j, l)
                nl = (l + NBUF - 1) % NBUF
                if l == 0:
                    @pl.when(g > 0)
                    def _():
                        scatter_wait(j - 1, nl)
                    @pl.when(j + NBUF - 1 < K)
                    def _():
                        gather_start(j + NBUF - 1, nl)
                else:
                    scatter_wait(j - 1, nl)
                    @pl.when(j + NBUF - 1 < K)
                    def _():
                        gather_start(j + NBUF - 1, nl)
            return carry

        lax.fori_loop(0, K // NBUF, group, 0)
        scatter_wait(K - 1, NBUF - 1)
        plsc.subcore_barrier()
        pltpu.sync_copy(acc.at[pl.ds(base, ROWS_PER_TILE)],
                        out_hbm.at[pl.ds(off, ROWS_PER_TILE)])


def _dis_block(deg_blk, block_id):
    cnt = jnp.sum(deg_blk, axis=1, keepdims=True) * (1.0 / DEG_W) + 1.0
    rows = lax.broadcasted_iota(jnp.int32, (R, 1), 0) + block_id * R
    return jnp.where(rows < N, lax.rsqrt(cnt), 0.0)


def _tc_enc_body(t_first, x_ref, we_ref, be_ref, wm_ref, wt_ref, deg_ref,
                 h_ref, y_ref):
    b = pl.program_id(0)
    h = jnp.maximum(
        jnp.dot(x_ref[...], we_ref[...], preferred_element_type=jnp.float32)
        + be_ref[...], 0.0)
    h_ref[...] = h
    dis = _dis_block(deg_ref[...], b)
    y = (jnp.dot(h, wm_ref[...], preferred_element_type=jnp.float32)
         + t_first * wt_ref[...]) * dis
    for q in range(4):
        y_ref[q] = y[:, q * QW:(q + 1) * QW]


def _tc_mid_body(t_next, h_ref, agg_ref, deg_ref, bg_ref, wm_ref, wt_ref,
                 ho_ref, yo_ref):
    b = pl.program_id(0)
    dis = _dis_block(deg_ref[...], b)
    aggcat = jnp.concatenate([agg_ref[q] for q in range(4)], axis=1)
    conv = dis * aggcat + bg_ref[...]
    hn = 0.5 * h_ref[...] + 0.5 * jnp.maximum(conv, 0.0)
    ho_ref[...] = hn
    y = (jnp.dot(hn, wm_ref[...], preferred_element_type=jnp.float32)
         + t_next * wt_ref[...]) * dis
    for q in range(4):
        yo_ref[q] = y[:, q * QW:(q + 1) * QW]


def _tc_last_body(h_ref, agg_ref, deg_ref, bg_ref, wd_ref, bd_ref, o_ref):
    b = pl.program_id(0)
    dis = _dis_block(deg_ref[...], b)
    aggcat = jnp.concatenate([agg_ref[q] for q in range(4)], axis=1)
    conv = dis * aggcat + bg_ref[...]
    hn = 0.5 * h_ref[...] + 0.5 * jnp.maximum(conv, 0.0)
    o_ref[...] = (jnp.dot(hn, wd_ref[...], preferred_element_type=jnp.float32)
                  + bd_ref[...])


_ROWB = pl.BlockSpec((R, D_H), lambda b: (b, 0))
_ROWB128 = pl.BlockSpec((R, 128), lambda b: (b, 0))
_Y_B = pl.BlockSpec((4, R, QW), lambda b: (0, b, 0))
_DEG_B = pl.BlockSpec((R, DEG_W), lambda b: (b, 0))
_FULL = lambda shape: pl.BlockSpec(shape, lambda b: tuple(0 for _ in shape))


def _tc_enc(t_first):
    return pl.pallas_call(
        functools.partial(_tc_enc_body, t_first),
        grid=(GRID,),
        in_specs=[_ROWB128, _FULL((D_IN, D_H)), _FULL((1, D_H)),
                  _FULL((D_H, D_H)), _FULL((1, D_H)), _DEG_B],
        out_specs=[_ROWB, _Y_B],
        out_shape=[jax.ShapeDtypeStruct((NP, D_H), jnp.float32),
                   jax.ShapeDtypeStruct((4, NP, QW), jnp.float32)],
    )


def _tc_mid(t_next):
    return pl.pallas_call(
        functools.partial(_tc_mid_body, t_next),
        grid=(GRID,),
        in_specs=[_ROWB, _Y_B, _DEG_B, _FULL((1, D_H)),
                  _FULL((D_H, D_H)), _FULL((1, D_H))],
        out_specs=[_ROWB, _Y_B],
        out_shape=[jax.ShapeDtypeStruct((NP, D_H), jnp.float32),
                   jax.ShapeDtypeStruct((4, NP, QW), jnp.float32)],
    )


_tc_last = pl.pallas_call(
    _tc_last_body,
    grid=(GRID,),
    in_specs=[_ROWB, _Y_B, _DEG_B, _FULL((1, D_H)),
              _FULL((D_H, D_OUT)), _FULL((1, D_OUT))],
    out_specs=_ROWB128,
    out_shape=jax.ShapeDtypeStruct((NP, D_OUT), jnp.float32),
)


def kernel(x, edge_index, W_enc, b_enc, W_gc, b_gc, W_dec, b_dec):
    src = edge_index[0].astype(jnp.int32)
    dst = edge_index[1].astype(jnp.int32)
    pad = N + (jnp.arange(EP - E, dtype=jnp.int32) % (NP - N))
    src_p = jnp.concatenate([src, pad])
    dst_p = jnp.concatenate([dst, pad]).reshape(NSUB, K, CHUNK)
    src_4 = jnp.stack([src_p, src_p + NP, src_p + 2 * NP, src_p + 3 * NP])
    src_4 = src_4.reshape(2, 2, NSUB, K, CHUNK)

    deg16 = _deg_kernel(dst_p)

    x_p = jnp.pad(x, ((0, NP - N), (0, 0)))
    Wm = W_gc[:D_H]
    wt = W_gc[D_H:D_H + 1]
    be = b_enc.reshape(1, D_H)
    bg = b_gc.reshape(1, D_H)
    bd = b_dec.reshape(1, D_OUT)

    num_iter = len(SCHEDULE)
    h, y = _tc_enc(1.0 / num_iter)(x_p, W_enc, be, Wm, wt, deg16)
    for it in range(1, num_iter):
        agg = _agg_kernel(y.reshape(4 * NP, QW), src_4, dst_p)
        h, y = _tc_mid((it + 1.0) / num_iter)(
            h, agg.reshape(4, NP, QW), deg16, bg, Wm, wt)
    agg = _agg_kernel(y.reshape(4 * NP, QW), src_4, dst_p)
    out = _tc_last(h, agg.reshape(4, NP, QW), deg16, bg, W_dec, bd)
    return out[:N]


# speedup vs baseline: 16.6501x; 1.0047x over previous
import functools

import jax
import jax.numpy as jnp
from jax import lax
from jax.experimental import pallas as pl
from jax.experimental.pallas import tpu as pltpu
from jax.experimental.pallas import tpu_sc as plsc

N = 10000
NP = 10240
E = 320000
D_IN = 128
D_H = 256
D_OUT = 128
NSUB = 16
CHUNK = 128
K = 162
EP = NSUB * K * CHUNK
ROWS_PER_TILE = NP // NSUB
R = 512
GRID = NP // R
SCHEDULE = (0.5, 0.5, 0.5, 0.5)
DEG_W = 16

_mesh = plsc.VectorSubcoreMesh(core_axis_name="c", subcore_axis_name="s")


@functools.partial(
    pl.kernel,
    mesh=_mesh,
    out_type=jax.ShapeDtypeStruct((NP, DEG_W), jnp.float32),
    scratch_types=[
        pltpu.VMEM((K, CHUNK), jnp.int32),
        pltpu.VMEM((CHUNK, DEG_W), jnp.float32),
        pltpu.VMEM((ROWS_PER_TILE, DEG_W), jnp.float32),
        pltpu.VMEM_SHARED((NP, DEG_W), jnp.float32),
    ],
    compiler_params=pltpu.CompilerParams(use_tc_tiling_on_sc=False),
)
def _deg_kernel(dst_hbm, out_hbm, dst_v, ones_v, zero_v, hist):
    c = lax.axis_index("c")
    s = lax.axis_index("s")
    pltpu.sync_copy(dst_hbm.at[s, :, :], dst_v)

    def fill_ones(i, carry):
        ones_v[i] = jnp.ones((DEG_W,), jnp.float32)
        return carry

    lax.fori_loop(0, CHUNK, fill_ones, 0)

    def fill_zero(i, carry):
        zero_v[i] = jnp.zeros((DEG_W,), jnp.float32)
        return carry

    lax.fori_loop(0, ROWS_PER_TILE, fill_zero, 0)
    pltpu.sync_copy(zero_v, hist.at[pl.ds(s * ROWS_PER_TILE, ROWS_PER_TILE)])
    plsc.subcore_barrier()

    def step(j, carry):
        pltpu.sync_copy(ones_v, hist.at[dst_v.at[j]], add=True)
        return carry

    lax.fori_loop(0, K, step, 0)
    plsc.subcore_barrier()
    half = NP // 2
    per = half // NSUB
    wb = c * half + s * per
    pltpu.sync_copy(hist.at[pl.ds(wb, per)], out_hbm.at[pl.ds(wb, per)])


QW = 64


NBUF = 3


@functools.partial(
    pl.kernel,
    mesh=_mesh,
    out_type=jax.ShapeDtypeStruct((4, NP, QW), jnp.float32),
    scratch_types=[
        pltpu.VMEM((K, CHUNK), jnp.int32),
        pltpu.VMEM((K, CHUNK), jnp.int32),
        [pltpu.VMEM((CHUNK, QW), jnp.float32) for _ in range(NBUF)],
        pltpu.VMEM_SHARED((NP, QW), jnp.float32),
        [pltpu.SemaphoreType.DMA for _ in range(NBUF)],
        [pltpu.SemaphoreType.DMA for _ in range(NBUF)],
    ],
    compiler_params=pltpu.CompilerParams(use_tc_tiling_on_sc=False),
)
def _agg_kernel(y_hbm, src_hbm, dst_hbm, out_hbm, src_v, dst_v,
                rows, acc, gsem, ssem):
    c = lax.axis_index("c")
    s = lax.axis_index("s")
    pltpu.sync_copy(src_hbm.at[s, :, :], src_v)
    pltpu.sync_copy(dst_hbm.at[s, :, :], dst_v)
    base = s * ROWS_PER_TILE

    for p in (0, 1):
        q = 2 * c + p
        yq = y_hbm.at[q, :, :]
        pltpu.sync_copy(y_hbm.at[q, pl.ds(base, ROWS_PER_TILE), :],
                        acc.at[pl.ds(base, ROWS_PER_TILE)])
        plsc.subcore_barrier()

        def gather_start(j, l):
            pltpu.async_copy(yq.at[src_v.at[j]], rows[l], gsem[l])

        def gather_wait(j, l):
            pltpu.make_async_copy(yq.at[src_v.at[j]], rows[l], gsem[l]).wait()

        def scatter_start(j, l):
            pltpu.async_copy(rows[l], acc.at[dst_v.at[j]], ssem[l], add=True)

        def scatter_wait(j, l):
            pltpu.make_async_copy(rows[l], acc.at[dst_v.at[j]], ssem[l]).wait()

        for l in range(NBUF - 1):
            gather_start(l, l)

        def group(g, carry):
            for l in range(NBUF):
                j = NBUF * g + l
                gather_wait(j, l)
                scatter_start(j, l)
                nl = (l + NBUF - 1) % NBUF
                if l == 0:
                    @pl.when(g > 0)
                    def _():
                        scatter_wait(j - 1, nl)
                    @pl.when(j + NBUF - 1 < K)
                    def _():
                        gather_start(j + NBUF - 1, nl)
                else:
                    scatter_wait(j - 1, nl)
                    @pl.when(j + NBUF - 1 < K)
                    def _():
                        gather_start(j + NBUF - 1, nl)
            return carry

        lax.fori_loop(0, K // NBUF, group, 0)
        scatter_wait(K - 1, NBUF - 1)
        plsc.subcore_barrier()
        pltpu.sync_copy(acc.at[pl.ds(base, ROWS_PER_TILE)],
                        out_hbm.at[q, pl.ds(base, ROWS_PER_TILE), :])


def _dis_block(deg_blk, block_id):
    cnt = jnp.sum(deg_blk, axis=1, keepdims=True) * (1.0 / DEG_W) + 1.0
    rows = lax.broadcasted_iota(jnp.int32, (R, 1), 0) + block_id * R
    return jnp.where(rows < N, lax.rsqrt(cnt), 0.0)


def _tc_enc_body(t_first, x_ref, we_ref, be_ref, wm_ref, wt_ref, deg_ref,
                 h_ref, y_ref):
    b = pl.program_id(0)
    h = jnp.maximum(
        jnp.dot(x_ref[...], we_ref[...], preferred_element_type=jnp.float32)
        + be_ref[...], 0.0)
    h_ref[...] = h
    dis = _dis_block(deg_ref[...], b)
    y = (jnp.dot(h, wm_ref[...], preferred_element_type=jnp.float32)
         + t_first * wt_ref[...]) * dis
    for q in range(4):
        y_ref[q] = y[:, q * QW:(q + 1) * QW]


def _tc_mid_body(t_next, h_ref, agg_ref, deg_ref, bg_ref, wm_ref, wt_ref,
                 ho_ref, yo_ref):
    b = pl.program_id(0)
    dis = _dis_block(deg_ref[...], b)
    aggcat = jnp.concatenate([agg_ref[q] for q in range(4)], axis=1)
    conv = dis * aggcat + bg_ref[...]
    hn = 0.5 * h_ref[...] + 0.5 * jnp.maximum(conv, 0.0)
    ho_ref[...] = hn
    y = (jnp.dot(hn, wm_ref[...], preferred_element_type=jnp.float32)
         + t_next * wt_ref[...]) * dis
    for q in range(4):
        yo_ref[q] = y[:, q * QW:(q + 1) * QW]


def _tc_last_body(h_ref, agg_ref, deg_ref, bg_ref, wd_ref, bd_ref, o_ref):
    b = pl.program_id(0)
    dis = _dis_block(deg_ref[...], b)
    aggcat = jnp.concatenate([agg_ref[q] for q in range(4)], axis=1)
    conv = dis * aggcat + bg_ref[...]
    hn = 0.5 * h_ref[...] + 0.5 * jnp.maximum(conv, 0.0)
    o_ref[...] = (jnp.dot(hn, wd_ref[...], preferred_element_type=jnp.float32)
                  + bd_ref[...])


_ROWB = pl.BlockSpec((R, D_H), lambda b: (b, 0))
_ROWB128 = pl.BlockSpec((R, 128), lambda b: (b, 0))
_Y_B = pl.BlockSpec((4, R, QW), lambda b: (0, b, 0))
_DEG_B = pl.BlockSpec((R, DEG_W), lambda b: (b, 0))
_FULL = lambda shape: pl.BlockSpec(shape, lambda b: tuple(0 for _ in shape))


def _tc_enc(t_first):
    return pl.pallas_call(
        functools.partial(_tc_enc_body, t_first),
        grid=(GRID,),
        in_specs=[_ROWB128, _FULL((D_IN, D_H)), _FULL((1, D_H)),
                  _FULL((D_H, D_H)), _FULL((1, D_H)), _DEG_B],
        out_specs=[_ROWB, _Y_B],
        out_shape=[jax.ShapeDtypeStruct((NP, D_H), jnp.float32),
                   jax.ShapeDtypeStruct((4, NP, QW), jnp.float32)],
    )


def _tc_mid(t_next):
    return pl.pallas_call(
        functools.partial(_tc_mid_body, t_next),
        grid=(GRID,),
        in_specs=[_ROWB, _Y_B, _DEG_B, _FULL((1, D_H)),
                  _FULL((D_H, D_H)), _FULL((1, D_H))],
        out_specs=[_ROWB, _Y_B],
        out_shape=[jax.ShapeDtypeStruct((NP, D_H), jnp.float32),
                   jax.ShapeDtypeStruct((4, NP, QW), jnp.float32)],
    )


_tc_last = pl.pallas_call(
    _tc_last_body,
    grid=(GRID,),
    in_specs=[_ROWB, _Y_B, _DEG_B, _FULL((1, D_H)),
              _FULL((D_H, D_OUT)), _FULL((1, D_OUT))],
    out_specs=_ROWB128,
    out_shape=jax.ShapeDtypeStruct((NP, D_OUT), jnp.float32),
)


def kernel(x, edge_index, W_enc, b_enc, W_gc, b_gc, W_dec, b_dec):
    src = edge_index[0].astype(jnp.int32)
    dst = edge_index[1].astype(jnp.int32)
    pad = N + (jnp.arange(EP - E, dtype=jnp.int32) % (NP - N))
    src_p = jnp.concatenate([src, pad])
    dst_p = jnp.concatenate([dst, pad]).reshape(NSUB, K, CHUNK)
    src_p = src_p.reshape(NSUB, K, CHUNK)

    deg16 = _deg_kernel(dst_p)

    x_p = jnp.pad(x, ((0, NP - N), (0, 0)))
    Wm = W_gc[:D_H]
    wt = W_gc[D_H:D_H + 1]
    be = b_enc.reshape(1, D_H)
    bg = b_gc.reshape(1, D_H)
    bd = b_dec.reshape(1, D_OUT)

    num_iter = len(SCHEDULE)
    h, y = _tc_enc(1.0 / num_iter)(x_p, W_enc, be, Wm, wt, deg16)
    for it in range(1, num_iter):
        agg = _agg_kernel(y, src_p, dst_p)
        h, y = _tc_mid((it + 1.0) / num_iter)(h, agg, deg16, bg, Wm, wt)
    agg = _agg_kernel(y, src_p, dst_p)
    out = _tc_last(h, agg, deg16, bg, W_dec, bd)
    return out[:N]


# speedup vs baseline: 16.7173x; 1.0040x over previous
import functools

import jax
import jax.numpy as jnp
from jax import lax
from jax.experimental import pallas as pl
from jax.experimental.pallas import tpu as pltpu
from jax.experimental.pallas import tpu_sc as plsc

N = 10000
NP = 10240
E = 320000
D_IN = 128
D_H = 256
D_OUT = 128
NSUB = 16
CHUNK = 128
K = 162
EP = NSUB * K * CHUNK
ROWS_PER_TILE = NP // NSUB
R = 512
GRID = NP // R
SCHEDULE = (0.5, 0.5, 0.5, 0.5)
DEG_W = 16

_mesh = plsc.VectorSubcoreMesh(core_axis_name="c", subcore_axis_name="s")


@functools.partial(
    pl.kernel,
    mesh=_mesh,
    out_type=jax.ShapeDtypeStruct((NP, DEG_W), jnp.float32),
    scratch_types=[
        pltpu.VMEM((K, CHUNK), jnp.int32),
        pltpu.VMEM((CHUNK, DEG_W), jnp.float32),
        pltpu.VMEM((ROWS_PER_TILE, DEG_W), jnp.float32),
        pltpu.VMEM_SHARED((NP, DEG_W), jnp.float32),
    ],
    compiler_params=pltpu.CompilerParams(use_tc_tiling_on_sc=False),
)
def _deg_kernel(dst_hbm, out_hbm, dst_v, ones_v, zero_v, hist):
    c = lax.axis_index("c")
    s = lax.axis_index("s")
    pltpu.sync_copy(dst_hbm.at[s, :, :], dst_v)

    def fill_ones(i, carry):
        ones_v[i] = jnp.ones((DEG_W,), jnp.float32)
        return carry

    lax.fori_loop(0, CHUNK, fill_ones, 0)

    def fill_zero(i, carry):
        zero_v[i] = jnp.zeros((DEG_W,), jnp.float32)
        return carry

    lax.fori_loop(0, ROWS_PER_TILE, fill_zero, 0)
    pltpu.sync_copy(zero_v, hist.at[pl.ds(s * ROWS_PER_TILE, ROWS_PER_TILE)])
    plsc.subcore_barrier()

    def step(j, carry):
        pltpu.sync_copy(ones_v, hist.at[dst_v.at[j]], add=True)
        return carry

    lax.fori_loop(0, K, step, 0)
    plsc.subcore_barrier()
    half = NP // 2
    per = half // NSUB
    wb = c * half + s * per
    pltpu.sync_copy(hist.at[pl.ds(wb, per)], out_hbm.at[pl.ds(wb, per)])


QW = 64


NBUF = 3


@functools.partial(
    pl.kernel,
    mesh=_mesh,
    out_type=jax.ShapeDtypeStruct((4, NP, QW), jnp.float32),
    scratch_types=[
        pltpu.VMEM((K, CHUNK), jnp.int32),
        pltpu.VMEM((K, CHUNK), jnp.int32),
        [pltpu.VMEM((CHUNK, QW), jnp.float32) for _ in range(NBUF)],
        pltpu.VMEM_SHARED((NP, QW), jnp.float32),
        [pltpu.SemaphoreType.DMA for _ in range(NBUF)],
        [pltpu.SemaphoreType.DMA for _ in range(NBUF)],
    ],
    compiler_params=pltpu.CompilerParams(use_tc_tiling_on_sc=False),
)
def _agg_kernel(y_hbm, src_hbm, dst_hbm, out_hbm, src_v, dst_v,
                rows, acc, gsem, ssem):
    c = lax.axis_index("c")
    s = lax.axis_index("s")
    pltpu.sync_copy(src_hbm.at[s, :, :], src_v)
    pltpu.sync_copy(dst_hbm.at[s, :, :], dst_v)
    base = s * ROWS_PER_TILE

    for p in (0, 1):
        q = 2 * c + p
        yq = y_hbm.at[q, :, :]
        pltpu.sync_copy(y_hbm.at[q, pl.ds(base, ROWS_PER_TILE), :],
                        acc.at[pl.ds(base, ROWS_PER_TILE)])
        plsc.subcore_barrier()

        def gather_start(j, l):
            pltpu.async_copy(yq.at[src_v.at[j]], rows[l], gsem[l])

        def gather_wait(j, l):
            pltpu.make_async_copy(yq.at[src_v.at[j]], rows[l], gsem[l]).wait()

        def scatter_start(j, l):
            pltpu.async_copy(rows[l], acc.at[dst_v.at[j]], ssem[l], add=True)

        def scatter_wait(j, l):
            pltpu.make_async_copy(rows[l], acc.at[dst_v.at[j]], ssem[l]).wait()

        for l in range(NBUF - 1):
            gather_start(l, l)

        def group(g, carry):
            for l in range(NBUF):
                j = NBUF * g + l
                gather_wait(j, l)
                scatter_start(j, l)
                nl = (l + NBUF - 1) % NBUF
                if l == 0:
                    @pl.when(g > 0)
                    def _():
                        scatter_wait(j - 1, nl)
                    @pl.when(j + NBUF - 1 < K)
                    def _():
                        gather_start(j + NBUF - 1, nl)
                else:
                    scatter_wait(j - 1, nl)
                    @pl.when(j + NBUF - 1 < K)
                    def _():
                        gather_start(j + NBUF - 1, nl)
            return carry

        lax.fori_loop(0, K // NBUF, group, 0)
        scatter_wait(K - 1, NBUF - 1)
        plsc.subcore_barrier()
        pltpu.sync_copy(acc.at[pl.ds(base, ROWS_PER_TILE)],
                        out_hbm.at[q, pl.ds(base, ROWS_PER_TILE), :])


def _dis_block(deg_blk, block_id):
    cnt = jnp.sum(deg_blk, axis=1, keepdims=True) * (1.0 / DEG_W) + 1.0
    rows = lax.broadcasted_iota(jnp.int32, (R, 1), 0) + block_id * R
    return jnp.where(rows < N, lax.rsqrt(cnt), 0.0)


def _tc_enc_body(t_first, x_ref, we_ref, be_ref, wm_ref, wt_ref, deg_ref,
                 h_ref, y_ref):
    b = pl.program_id(0)
    h = jnp.maximum(
        jnp.dot(x_ref[...], we_ref[...], preferred_element_type=jnp.float32)
        + be_ref[...], 0.0)
    h_ref[...] = h
    dis = _dis_block(deg_ref[...], b)
    y = (jnp.dot(h, wm_ref[...], preferred_element_type=jnp.float32)
         + t_first * wt_ref[...]) * dis
    for q in range(4):
        y_ref[q] = y[:, q * QW:(q + 1) * QW]


def _tc_mid_body(t_next, h_ref, agg_ref, deg_ref, bg_ref, wm_ref, wt_ref,
                 ho_ref, yo_ref):
    b = pl.program_id(0)
    dis = _dis_block(deg_ref[...], b)
    aggcat = jnp.concatenate([agg_ref[q] for q in range(4)], axis=1)
    conv = dis * aggcat + bg_ref[...]
    hn = 0.5 * h_ref[...] + 0.5 * jnp.maximum(conv, 0.0)
    ho_ref[...] = hn
    y = (jnp.dot(hn, wm_ref[...], preferred_element_type=jnp.float32)
         + t_next * wt_ref[...]) * dis
    for q in range(4):
        yo_ref[q] = y[:, q * QW:(q + 1) * QW]


def _tc_last_body(h_ref, agg_ref, deg_ref, bg_ref, wd_ref, bd_ref, o_ref):
    b = pl.program_id(0)
    dis = _dis_block(deg_ref[...], b)
    aggcat = jnp.concatenate([agg_ref[q] for q in range(4)], axis=1)
    conv = dis * aggcat + bg_ref[...]
    hn = 0.5 * h_ref[...] + 0.5 * jnp.maximum(conv, 0.0)
    o_ref[...] = (jnp.dot(hn, wd_ref[...], preferred_element_type=jnp.float32)
                  + bd_ref[...])


_ROWB = pl.BlockSpec((R, D_H), lambda b: (b, 0))
_ROWB128 = pl.BlockSpec((R, 128), lambda b: (b, 0))
_Y_B = pl.BlockSpec((4, R, QW), lambda b: (0, b, 0))
_DEG_B = pl.BlockSpec((R, DEG_W), lambda b: (b, 0))
_FULL = lambda shape: pl.BlockSpec(shape, lambda b: tuple(0 for _ in shape))


def _tc_enc(t_first):
    return pl.pallas_call(
        functools.partial(_tc_enc_body, t_first),
        grid=(GRID,),
        in_specs=[_ROWB128, _FULL((D_IN, D_H)), _FULL((1, D_H)),
                  _FULL((D_H, D_H)), _FULL((1, D_H)), _DEG_B],
        out_specs=[_ROWB, _Y_B],
        out_shape=[jax.ShapeDtypeStruct((N, D_H), jnp.float32),
                   jax.ShapeDtypeStruct((4, NP, QW), jnp.float32)],
    )


def _tc_mid(t_next):
    return pl.pallas_call(
        functools.partial(_tc_mid_body, t_next),
        grid=(GRID,),
        in_specs=[_ROWB, _Y_B, _DEG_B, _FULL((1, D_H)),
                  _FULL((D_H, D_H)), _FULL((1, D_H))],
        out_specs=[_ROWB, _Y_B],
        out_shape=[jax.ShapeDtypeStruct((N, D_H), jnp.float32),
                   jax.ShapeDtypeStruct((4, NP, QW), jnp.float32)],
    )


_tc_last = pl.pallas_call(
    _tc_last_body,
    grid=(GRID,),
    in_specs=[_ROWB, _Y_B, _DEG_B, _FULL((1, D_H)),
              _FULL((D_H, D_OUT)), _FULL((1, D_OUT))],
    out_specs=_ROWB128,
    out_shape=jax.ShapeDtypeStruct((N, D_OUT), jnp.float32),
)


def kernel(x, edge_index, W_enc, b_enc, W_gc, b_gc, W_dec, b_dec):
    src = edge_index[0].astype(jnp.int32)
    dst = edge_index[1].astype(jnp.int32)
    pad = N + (jnp.arange(EP - E, dtype=jnp.int32) % (NP - N))
    src_p = jnp.concatenate([src, pad])
    dst_p = jnp.concatenate([dst, pad]).reshape(NSUB, K, CHUNK)
    src_p = src_p.reshape(NSUB, K, CHUNK)

    deg16 = _deg_kernel(dst_p)

    Wm = W_gc[:D_H]
    wt = W_gc[D_H:D_H + 1]
    be = b_enc.reshape(1, D_H)
    bg = b_gc.reshape(1, D_H)
    bd = b_dec.reshape(1, D_OUT)

    num_iter = len(SCHEDULE)
    h, y = _tc_enc(1.0 / num_iter)(x, W_enc, be, Wm, wt, deg16)
    for it in range(1, num_iter):
        agg = _agg_kernel(y, src_p, dst_p)
        h, y = _tc_mid((it + 1.0) / num_iter)(h, agg, deg16, bg, Wm, wt)
    agg = _agg_kernel(y, src_p, dst_p)
    return _tc_last(h, agg, deg16, bg, W_dec, bd)


# speedup vs baseline: 16.9040x; 1.0112x over previous
import functools

import jax
import jax.numpy as jnp
from jax import lax
from jax.experimental import pallas as pl
from jax.experimental.pallas import tpu as pltpu
from jax.experimental.pallas import tpu_sc as plsc

N = 10000
NP = 10240
E = 320000
D_IN = 128
D_H = 256
D_OUT = 128
NSUB = 16
CHUNK = 128
K = 159
EP = NSUB * K * CHUNK
ROWS_PER_TILE = NP // NSUB
R = 512
GRID = NP // R
SCHEDULE = (0.5, 0.5, 0.5, 0.5)
DEG_W = 16

_mesh = plsc.VectorSubcoreMesh(core_axis_name="c", subcore_axis_name="s")


@functools.partial(
    pl.kernel,
    mesh=_mesh,
    out_type=jax.ShapeDtypeStruct((NP, DEG_W), jnp.float32),
    scratch_types=[
        pltpu.VMEM((K, CHUNK), jnp.int32),
        pltpu.VMEM((CHUNK, DEG_W), jnp.float32),
        pltpu.VMEM((ROWS_PER_TILE, DEG_W), jnp.float32),
        pltpu.VMEM_SHARED((NP, DEG_W), jnp.float32),
    ],
    compiler_params=pltpu.CompilerParams(use_tc_tiling_on_sc=False),
)
def _deg_kernel(dst_hbm, out_hbm, dst_v, ones_v, zero_v, hist):
    c = lax.axis_index("c")
    s = lax.axis_index("s")
    pltpu.sync_copy(dst_hbm.at[s, :, :], dst_v)

    def fill_ones(i, carry):
        ones_v[i] = jnp.ones((DEG_W,), jnp.float32)
        return carry

    lax.fori_loop(0, CHUNK, fill_ones, 0)

    def fill_zero(i, carry):
        zero_v[i] = jnp.zeros((DEG_W,), jnp.float32)
        return carry

    lax.fori_loop(0, ROWS_PER_TILE, fill_zero, 0)
    pltpu.sync_copy(zero_v, hist.at[pl.ds(s * ROWS_PER_TILE, ROWS_PER_TILE)])
    plsc.subcore_barrier()

    def step(j, carry):
        pltpu.sync_copy(ones_v, hist.at[dst_v.at[j]], add=True)
        return carry

    lax.fori_loop(0, K, step, 0)
    plsc.subcore_barrier()
    half = NP // 2
    per = half // NSUB
    wb = c * half + s * per
    pltpu.sync_copy(hist.at[pl.ds(wb, per)], out_hbm.at[pl.ds(wb, per)])


QW = 64


NBUF = 3


@functools.partial(
    pl.kernel,
    mesh=_mesh,
    out_type=jax.ShapeDtypeStruct((4, NP, QW), jnp.float32),
    scratch_types=[
        pltpu.VMEM((K, CHUNK), jnp.int32),
        pltpu.VMEM((K, CHUNK), jnp.int32),
        [pltpu.VMEM((CHUNK, QW), jnp.float32) for _ in range(NBUF)],
        pltpu.VMEM_SHARED((NP, QW), jnp.float32),
        [pltpu.SemaphoreType.DMA for _ in range(NBUF)],
        [pltpu.SemaphoreType.DMA for _ in range(NBUF)],
    ],
    compiler_params=pltpu.CompilerParams(use_tc_tiling_on_sc=False),
)
def _agg_kernel(y_hbm, src_hbm, dst_hbm, out_hbm, src_v, dst_v,
                rows, acc, gsem, ssem):
    c = lax.axis_index("c")
    s = lax.axis_index("s")
    pltpu.sync_copy(src_hbm.at[s, :, :], src_v)
    pltpu.sync_copy(dst_hbm.at[s, :, :], dst_v)
    base = s * ROWS_PER_TILE

    for p in (0, 1):
        q = 2 * c + p
        yq = y_hbm.at[q, :, :]
        pltpu.sync_copy(y_hbm.at[q, pl.ds(base, ROWS_PER_TILE), :],
                        acc.at[pl.ds(base, ROWS_PER_TILE)])
        plsc.subcore_barrier()

        def gather_start(j, l):
            pltpu.async_copy(yq.at[src_v.at[j]], rows[l], gsem[l])

        def gather_wait(j, l):
            pltpu.make_async_copy(yq.at[src_v.at[j]], rows[l], gsem[l]).wait()

        def scatter_start(j, l):
            pltpu.async_copy(rows[l], acc.at[dst_v.at[j]], ssem[l], add=True)

        def scatter_wait(j, l):
            pltpu.make_async_copy(rows[l], acc.at[dst_v.at[j]], ssem[l]).wait()

        for l in range(NBUF - 1):
            gather_start(l, l)

        def group(g, carry):
            for l in range(NBUF):
                j = NBUF * g + l
                gather_wait(j, l)
                scatter_start(j, l)
                nl = (l + NBUF - 1) % NBUF
                if l == 0:
                    @pl.when(g > 0)
                    def _():
                        scatter_wait(j - 1, nl)
                    @pl.when(j + NBUF - 1 < K)
                    def _():
                        gather_start(j + NBUF - 1, nl)
                else:
                    scatter_wait(j - 1, nl)
                    @pl.when(j + NBUF - 1 < K)
                    def _():
                        gather_start(j + NBUF - 1, nl)
            return carry

        lax.fori_loop(0, K // NBUF, group, 0)
        scatter_wait(K - 1, NBUF - 1)
        plsc.subcore_barrier()
        pltpu.sync_copy(acc.at[pl.ds(base, ROWS_PER_TILE)],
                        out_hbm.at[q, pl.ds(base, ROWS_PER_TILE), :])


def _dis_block(deg_blk, block_id):
    cnt = jnp.sum(deg_blk, axis=1, keepdims=True) * (1.0 / DEG_W) + 1.0
    rows = lax.broadcasted_iota(jnp.int32, (R, 1), 0) + block_id * R
    return jnp.where(rows < N, lax.rsqrt(cnt), 0.0)


def _tc_enc_body(t_first, x_ref, we_ref, be_ref, wm_ref, wt_ref, deg_ref,
                 h_ref, y_ref):
    b = pl.program_id(0)
    h = jnp.maximum(
        jnp.dot(x_ref[...], we_ref[...], preferred_element_type=jnp.float32)
        + be_ref[...], 0.0)
    h_ref[...] = h
    dis = _dis_block(deg_ref[...], b)
    y = (jnp.dot(h, wm_ref[...], preferred_element_type=jnp.float32)
         + t_first * wt_ref[...]) * dis
    for q in range(4):
        y_ref[q] = y[:, q * QW:(q + 1) * QW]


def _tc_mid_body(t_next, h_ref, agg_ref, deg_ref, bg_ref, wm_ref, wt_ref,
                 ho_ref, yo_ref):
    b = pl.program_id(0)
    dis = _dis_block(deg_ref[...], b)
    aggcat = jnp.concatenate([agg_ref[q] for q in range(4)], axis=1)
    conv = dis * aggcat + bg_ref[...]
    hn = 0.5 * h_ref[...] + 0.5 * jnp.maximum(conv, 0.0)
    ho_ref[...] = hn
    y = (jnp.dot(hn, wm_ref[...], preferred_element_type=jnp.float32)
         + t_next * wt_ref[...]) * dis
    for q in range(4):
        yo_ref[q] = y[:, q * QW:(q + 1) * QW]


def _tc_last_body(h_ref, agg_ref, deg_ref, bg_ref, wd_ref, bd_ref, o_ref):
    b = pl.program_id(0)
    dis = _dis_block(deg_ref[...], b)
    aggcat = jnp.concatenate([agg_ref[q] for q in range(4)], axis=1)
    conv = dis * aggcat + bg_ref[...]
    hn = 0.5 * h_ref[...] + 0.5 * jnp.maximum(conv, 0.0)
    o_ref[...] = (jnp.dot(hn, wd_ref[...], preferred_element_type=jnp.float32)
                  + bd_ref[...])


_ROWB = pl.BlockSpec((R, D_H), lambda b: (b, 0))
_ROWB128 = pl.BlockSpec((R, 128), lambda b: (b, 0))
_Y_B = pl.BlockSpec((4, R, QW), lambda b: (0, b, 0))
_DEG_B = pl.BlockSpec((R, DEG_W), lambda b: (b, 0))
_FULL = lambda shape: pl.BlockSpec(shape, lambda b: tuple(0 for _ in shape))


def _tc_enc(t_first):
    return pl.pallas_call(
        functools.partial(_tc_enc_body, t_first),
        grid=(GRID,),
        in_specs=[_ROWB128, _FULL((D_IN, D_H)), _FULL((1, D_H)),
                  _FULL((D_H, D_H)), _FULL((1, D_H)), _DEG_B],
        out_specs=[_ROWB, _Y_B],
        out_shape=[jax.ShapeDtypeStruct((N, D_H), jnp.float32),
                   jax.ShapeDtypeStruct((4, NP, QW), jnp.float32)],
    )


def _tc_mid(t_next):
    return pl.pallas_call(
        functools.partial(_tc_mid_body, t_next),
        grid=(GRID,),
        in_specs=[_ROWB, _Y_B, _DEG_B, _FULL((1, D_H)),
                  _FULL((D_H, D_H)), _FULL((1, D_H))],
        out_specs=[_ROWB, _Y_B],
        out_shape=[jax.ShapeDtypeStruct((N, D_H), jnp.float32),
                   jax.ShapeDtypeStruct((4, NP, QW), jnp.float32)],
    )


_tc_last = pl.pallas_call(
    _tc_last_body,
    grid=(GRID,),
    in_specs=[_ROWB, _Y_B, _DEG_B, _FULL((1, D_H)),
              _FULL((D_H, D_OUT)), _FULL((1, D_OUT))],
    out_specs=_ROWB128,
    out_shape=jax.ShapeDtypeStruct((N, D_OUT), jnp.float32),
)


def kernel(x, edge_index, W_enc, b_enc, W_gc, b_gc, W_dec, b_dec):
    src = edge_index[0].astype(jnp.int32)
    dst = edge_index[1].astype(jnp.int32)
    pad = N + (jnp.arange(EP - E, dtype=jnp.int32) % (NP - N))
    src_p = jnp.concatenate([src, pad])
    dst_p = jnp.concatenate([dst, pad]).reshape(NSUB, K, CHUNK)
    src_p = src_p.reshape(NSUB, K, CHUNK)

    deg16 = _deg_kernel(dst_p)

    Wm = W_gc[:D_H]
    wt = W_gc[D_H:D_H + 1]
    be = b_enc.reshape(1, D_H)
    bg = b_gc.reshape(1, D_H)
    bd = b_dec.reshape(1, D_OUT)

    num_iter = len(SCHEDULE)
    h, y = _tc_enc(1.0 / num_iter)(x, W_enc, be, Wm, wt, deg16)
    for it in range(1, num_iter):
        agg = _agg_kernel(y, src_p, dst_p)
        h, y = _tc_mid((it + 1.0) / num_iter)(h, agg, deg16, bg, Wm, wt)
    agg = _agg_kernel(y, src_p, dst_p)
    return _tc_last(h, agg, deg16, bg, W_dec, bd)
